# baseline (speedup 1.0000x reference)
import jax
import jax.numpy as jnp
from jax import lax
from jax.experimental import pallas as pl
from jax.experimental.pallas import tpu as pltpu
from jax.experimental.pallas import tpu_sc as plsc

B = 16
SEG = 16384
D = 64
NCLS = 16
T = B * SEG
VOCAB = 1000000

NC = 2
NS = 16
CHUNK = 128
ROWS_PER_TILE = SEG // 2 // CHUNK
NCHUNK = ROWS_PER_TILE
QS = D // 16

TW = 24576
NBLK = 21
PROWS = NBLK * TW


def _tr_body(lo_ref, hi_ref, out_ref):
    eye = jnp.eye(D, dtype=jnp.float32)
    dn = (((0,), (0,)), ((), ()))
    out_ref[:, 0:D] = lax.dot_general(
        lo_ref[...], eye, dn, preferred_element_type=jnp.float32)
    out_ref[:, D:2 * D] = lax.dot_general(
        hi_ref[...], eye, dn, preferred_element_type=jnp.float32)


_pack_table = pl.pallas_call(
    _tr_body,
    grid=(NBLK,),
    in_specs=[
        pl.BlockSpec((D, TW), lambda j: (0, j)),
        pl.BlockSpec((D, TW), lambda j: (0, jnp.minimum(NBLK + j,
                                                        (VOCAB - 1) // TW))),
    ],
    out_specs=pl.BlockSpec((TW, 2 * D), lambda j: (j, 0)),
    out_shape=jax.ShapeDtypeStruct((PROWS, 2 * D), jnp.float32),
    compiler_params=pltpu.CompilerParams(vmem_limit_bytes=100 * 1024 * 1024),
)


def _sc_body(phys2, text2, emb2, out, pidx_v, idx_v, buf0, buf1, acc_v,
             sem0, sem1):
    c = lax.axis_index("c")
    s = lax.axis_index("s")
    batch = c * (B // NC) + s // 2
    half = s % 2
    row_base = batch * (SEG // CHUNK) + half * ROWS_PER_TILE

    pltpu.sync_copy(phys2.at[pl.ds(row_base, ROWS_PER_TILE)], pidx_v)
    pltpu.sync_copy(text2.at[pl.ds(row_base, ROWS_PER_TILE)], idx_v)

    pltpu.async_copy(emb2.at[pidx_v.at[0]], buf0, sem0)
    pltpu.async_copy(emb2.at[pidx_v.at[1]], buf1, sem1)

    def acc_chunk(g, buf, accs):
        def row_body(r, a):
            a = list(a)
            pvec = idx_v[g, pl.ds(pl.multiple_of(r * 16, 16), 16)]
            for u in range(16):
                row = r * 16 + u
                off = pvec[u]
                for q in range(QS):
                    a[(u % 2) * QS + q] = a[(u % 2) * QS + q] + buf[
                        row, pl.ds(pl.multiple_of(off + q * 16, 16), 16)]
            return tuple(a)
        return lax.fori_loop(0, CHUNK // 16, row_body, accs)

    zero = jnp.zeros((16,), jnp.float32)
    accs0 = (zero,) * (2 * QS)

    def outer(i, accs):
        pltpu.make_async_copy(emb2.at[pidx_v.at[0]], buf0, sem0).wait()
        accs = acc_chunk(2 * i, buf0, accs)

        @pl.when(i < NCHUNK // 2 - 1)
        def _():
            pltpu.async_copy(emb2.at[pidx_v.at[2 * i + 2]], buf0, sem0)

        pltpu.make_async_copy(emb2.at[pidx_v.at[1]], buf1, sem1).wait()
        accs = acc_chunk(2 * i + 1, buf1, accs)

        @pl.when(i < NCHUNK // 2 - 1)
        def _():
            pltpu.async_copy(emb2.at[pidx_v.at[2 * i + 3]], buf1, sem1)

        return accs

    accs = lax.fori_loop(0, NCHUNK // 2, outer, accs0)

    for q in range(QS):
        acc_v[pl.ds(q * 16, 16)] = accs[q] + accs[QS + q]
    pltpu.sync_copy(acc_v, out.at[half, batch])


_sc_partials = pl.kernel(
    _sc_body,
    out_type=jax.ShapeDtypeStruct((2, B, D), jnp.float32),
    mesh=plsc.VectorSubcoreMesh(
        core_axis_name="c", subcore_axis_name="s", num_cores=NC,
        num_subcores=NS),
    scratch_types=[
        pltpu.VMEM((ROWS_PER_TILE, CHUNK), jnp.int32),
        pltpu.VMEM((ROWS_PER_TILE, CHUNK), jnp.int32),
        pltpu.VMEM((CHUNK, 2 * D), jnp.float32),
        pltpu.VMEM((CHUNK, 2 * D), jnp.float32),
        pltpu.VMEM((D,), jnp.float32),
        pltpu.SemaphoreType.DMA,
        pltpu.SemaphoreType.DMA,
    ],
)


def _tc_head(partials_ref, fc_ref, bias_ref, out_ref):
    pooled = (partials_ref[0] + partials_ref[1]) * (1.0 / SEG)
    out = lax.dot_general(
        pooled, fc_ref[...], (((1,), (1,)), ((), ())),
        preferred_element_type=jnp.float32)
    out_ref[...] = out + bias_ref[...]


_head = pl.pallas_call(
    _tc_head,
    out_shape=jax.ShapeDtypeStruct((B, NCLS), jnp.float32),
)


@jax.jit
def kernel(text, emb_weight, fc_weight, fc_bias):
    in_hi = text >= PROWS
    phys2 = jnp.where(in_hi, text - PROWS, text).reshape(T // CHUNK, CHUNK)
    off2 = jnp.where(in_hi, D, 0).astype(jnp.int32).reshape(
        T // CHUNK, CHUNK)
    embt = emb_weight.T
    emb2 = _pack_table(embt, embt)
    partials = _sc_partials(phys2, off2, emb2)
    return _head(partials, fc_weight, fc_bias.reshape(1, NCLS))


# speedup vs baseline: 1.0129x; 1.0129x over previous
import jax
import jax.numpy as jnp
from jax import lax
from jax.experimental import pallas as pl
from jax.experimental.pallas import tpu as pltpu
from jax.experimental.pallas import tpu_sc as plsc

B = 16
SEG = 16384
D = 64
NCLS = 16
T = B * SEG
VOCAB = 1000000

NC = 2
NS = 16
CHUNK = 128
ROWS_PER_TILE = SEG // 2 // CHUNK
NCHUNK = ROWS_PER_TILE
QS = D // 16

TW = 16384
NBLK = 31
PROWS = NBLK * TW


def _tr_body(lo_ref, hi_ref, out_ref):
    eye = jnp.eye(D, dtype=jnp.float32)
    dn = (((0,), (0,)), ((), ()))
    out_ref[:, 0:D] = lax.dot_general(
        lo_ref[...], eye, dn, preferred_element_type=jnp.float32)
    out_ref[:, D:2 * D] = lax.dot_general(
        hi_ref[...], eye, dn, preferred_element_type=jnp.float32)


_pack_table = pl.pallas_call(
    _tr_body,
    grid=(NBLK,),
    in_specs=[
        pl.BlockSpec((D, TW), lambda j: (0, j)),
        pl.BlockSpec((D, TW), lambda j: (0, jnp.minimum(NBLK + j,
                                                        (VOCAB - 1) // TW))),
    ],
    out_specs=pl.BlockSpec((TW, 2 * D), lambda j: (j, 0)),
    out_shape=jax.ShapeDtypeStruct((PROWS, 2 * D), jnp.float32),
)


def _sc_body(phys2, text2, emb2, out, pidx_v, idx_v, buf0, buf1, acc_v,
             sem0, sem1):
    c = lax.axis_index("c")
    s = lax.axis_index("s")
    batch = c * (B // NC) + s // 2
    half = s % 2
    row_base = batch * (SEG // CHUNK) + half * ROWS_PER_TILE

    pltpu.sync_copy(phys2.at[pl.ds(row_base, ROWS_PER_TILE)], pidx_v)
    pltpu.sync_copy(text2.at[pl.ds(row_base, ROWS_PER_TILE)], idx_v)

    pltpu.async_copy(emb2.at[pidx_v.at[0]], buf0, sem0)
    pltpu.async_copy(emb2.at[pidx_v.at[1]], buf1, sem1)

    def acc_chunk(g, buf, accs):
        def row_body(r, a):
            a = list(a)
            pvec = idx_v[g, pl.ds(pl.multiple_of(r * 16, 16), 16)]
            for u in range(16):
                row = r * 16 + u
                off = pvec[u]
                for q in range(QS):
                    a[(u % 2) * QS + q] = a[(u % 2) * QS + q] + buf[
                        row, pl.ds(pl.multiple_of(off + q * 16, 16), 16)]
            return tuple(a)
        return lax.fori_loop(0, CHUNK // 16, row_body, accs)

    zero = jnp.zeros((16,), jnp.float32)
    accs0 = (zero,) * (2 * QS)

    def outer(i, accs):
        pltpu.make_async_copy(emb2.at[pidx_v.at[0]], buf0, sem0).wait()
        accs = acc_chunk(2 * i, buf0, accs)

        @pl.when(i < NCHUNK // 2 - 1)
        def _():
            pltpu.async_copy(emb2.at[pidx_v.at[2 * i + 2]], buf0, sem0)

        pltpu.make_async_copy(emb2.at[pidx_v.at[1]], buf1, sem1).wait()
        accs = acc_chunk(2 * i + 1, buf1, accs)

        @pl.when(i < NCHUNK // 2 - 1)
        def _():
            pltpu.async_copy(emb2.at[pidx_v.at[2 * i + 3]], buf1, sem1)

        return accs

    accs = lax.fori_loop(0, NCHUNK // 2, outer, accs0)

    for q in range(QS):
        acc_v[pl.ds(q * 16, 16)] = accs[q] + accs[QS + q]
    pltpu.sync_copy(acc_v, out.at[half, batch])


_sc_partials = pl.kernel(
    _sc_body,
    out_type=jax.ShapeDtypeStruct((2, B, D), jnp.float32),
    mesh=plsc.VectorSubcoreMesh(
        core_axis_name="c", subcore_axis_name="s", num_cores=NC,
        num_subcores=NS),
    scratch_types=[
        pltpu.VMEM((ROWS_PER_TILE, CHUNK), jnp.int32),
        pltpu.VMEM((ROWS_PER_TILE, CHUNK), jnp.int32),
        pltpu.VMEM((CHUNK, 2 * D), jnp.float32),
        pltpu.VMEM((CHUNK, 2 * D), jnp.float32),
        pltpu.VMEM((D,), jnp.float32),
        pltpu.SemaphoreType.DMA,
        pltpu.SemaphoreType.DMA,
    ],
)


def _tc_head(partials_ref, fc_ref, bias_ref, out_ref):
    pooled = (partials_ref[0] + partials_ref[1]) * (1.0 / SEG)
    out = lax.dot_general(
        pooled, fc_ref[...], (((1,), (1,)), ((), ())),
        preferred_element_type=jnp.float32)
    out_ref[...] = out + bias_ref[...]


_head = pl.pallas_call(
    _tc_head,
    out_shape=jax.ShapeDtypeStruct((B, NCLS), jnp.float32),
)


@jax.jit
def kernel(text, emb_weight, fc_weight, fc_bias):
    in_hi = text >= PROWS
    phys2 = jnp.where(in_hi, text - PROWS, text).reshape(T // CHUNK, CHUNK)
    off2 = jnp.where(in_hi, D, 0).astype(jnp.int32).reshape(
        T // CHUNK, CHUNK)
    embt = emb_weight.T
    emb2 = _pack_table(embt, embt)
    partials = _sc_partials(phys2, off2, emb2)
    return _head(partials, fc_weight, fc_bias.reshape(1, NCLS))
